# baseline (speedup 1.0000x reference)
import jax
import jax.numpy as jnp
from jax.experimental import pallas as pl
from jax.experimental.pallas import tpu as pltpu

BLOCK_M = 10000


def _linear_block(x_ref, wt_ref, b_ref, o_ref):
    o_ref[...] = (
        jnp.dot(x_ref[...], wt_ref[...], preferred_element_type=jnp.float32)
        + b_ref[...]
    )


def kernel(x, W, b):
    n, k = x.shape
    c = W.shape[0]
    wt = W.T
    b2 = b.reshape(1, c)
    grid = (n // BLOCK_M,)
    return pl.pallas_call(
        _linear_block,
        grid=grid,
        in_specs=[
            pl.BlockSpec((BLOCK_M, k), lambda i: (i, 0)),
            pl.BlockSpec((k, c), lambda i: (0, 0)),
            pl.BlockSpec((1, c), lambda i: (0, 0)),
        ],
        out_specs=pl.BlockSpec((BLOCK_M, c), lambda i: (i, 0)),
        out_shape=jax.ShapeDtypeStruct((n, c), jnp.float32),
        compiler_params=pltpu.CompilerParams(
            dimension_semantics=("parallel",)
        ),
    )(x, wt, b2)


# speedup vs baseline: 1.0134x; 1.0134x over previous
import jax
import jax.numpy as jnp
from jax.experimental import pallas as pl
from jax.experimental.pallas import tpu as pltpu

CHUNK = 4000
SLOTS = 4


def _linear_pipeline(x_hbm, wt_ref, b_ref, o_hbm, xbuf, obuf, in_sem, out_sem):
    n = x_hbm.shape[0]
    nchunks = n // CHUNK

    def in_copy(c, slot):
        return pltpu.make_async_copy(
            x_hbm.at[pl.ds(c * CHUNK, CHUNK), :],
            xbuf.at[slot],
            in_sem.at[slot],
        )

    def out_copy(c, slot):
        return pltpu.make_async_copy(
            obuf.at[slot],
            o_hbm.at[pl.ds(c * CHUNK, CHUNK), :],
            out_sem.at[slot],
        )

    for s in range(SLOTS - 1):
        in_copy(s, s).start()

    def body(i, _):
        slot = jax.lax.rem(i, SLOTS)
        @pl.when(i >= SLOTS)
        def _():
            out_copy(i - SLOTS, slot).wait()

        nxt = i + SLOTS - 1

        @pl.when(nxt < nchunks)
        def _():
            in_copy(nxt, jax.lax.rem(nxt, SLOTS)).start()

        in_copy(i, slot).wait()
        obuf[slot] = (
            jnp.dot(xbuf[slot], wt_ref[...], preferred_element_type=jnp.float32)
            + b_ref[...]
        )
        out_copy(i, slot).start()
        return 0

    jax.lax.fori_loop(0, nchunks, body, 0)

    for s in range(min(SLOTS, nchunks)):
        c = nchunks - 1 - s
        out_copy(c, jax.lax.rem(c, SLOTS)).wait()


def kernel(x, W, b):
    n, k = x.shape
    c = W.shape[0]
    wt = W.T
    b2 = b.reshape(1, c)
    return pl.pallas_call(
        _linear_pipeline,
        in_specs=[
            pl.BlockSpec(memory_space=pl.ANY),
            pl.BlockSpec((k, c), lambda: (0, 0)),
            pl.BlockSpec((1, c), lambda: (0, 0)),
        ],
        out_specs=pl.BlockSpec(memory_space=pl.ANY),
        out_shape=jax.ShapeDtypeStruct((n, c), jnp.float32),
        scratch_shapes=[
            pltpu.VMEM((SLOTS, CHUNK, k), jnp.float32),
            pltpu.VMEM((SLOTS, CHUNK, c), jnp.float32),
            pltpu.SemaphoreType.DMA((SLOTS,)),
            pltpu.SemaphoreType.DMA((SLOTS,)),
        ],
    )(x, wt, b2)
